# Initial kernel scaffold; baseline (speedup 1.0000x reference)
import jax
import jax.numpy as jnp
from jax.experimental import pallas as pl


def kernel(x, weight):
    raise NotImplementedError("write your pallas kernel here")


# speedup vs baseline: 1.4801x; 1.4801x over previous
import functools

import jax
import jax.numpy as jnp
from jax import lax
from jax.experimental import pallas as pl
from jax.experimental.pallas import tpu as pltpu
from jax.experimental.pallas import tpu_sc as plsc

NUM_ROWS = 1_000_000
DIM = 32
B_TOTAL = 4096 * 200

_info = plsc.get_sparse_core_info()
NC, NS = _info.num_cores, _info.num_subcores
NW = NC * NS
PER_W = B_TOTAL // NW
CHUNK = 1600
N_CHUNKS = PER_W // CHUNK


def _make_gather():
    mesh = plsc.VectorSubcoreMesh(core_axis_name="c", subcore_axis_name="s")

    @functools.partial(
        pl.kernel,
        mesh=mesh,
        out_type=jax.ShapeDtypeStruct((B_TOTAL, DIM), jnp.float32),
        scratch_types=[
            pltpu.VMEM((CHUNK,), jnp.int32),
            pltpu.VMEM((CHUNK, DIM), jnp.float32),
            pltpu.SemaphoreType.DMA,
        ],
        compiler_params=pltpu.CompilerParams(use_tc_tiling_on_sc=False),
    )
    def gather(table_hbm, idx_hbm, out_hbm, idx_v, rows_v, sem):
        wid = lax.axis_index("s") * NC + lax.axis_index("c")
        w_base = wid * PER_W

        def body(i, _):
            base = w_base + i * CHUNK
            pltpu.sync_copy(idx_hbm.at[pl.ds(base, CHUNK)], idx_v)
            pltpu.async_copy(table_hbm.at[idx_v], rows_v, sem).wait()
            pltpu.sync_copy(rows_v, out_hbm.at[pl.ds(base, CHUNK)])
            return 0

        lax.fori_loop(0, N_CHUNKS, body, 0)

    return gather


_gather = _make_gather()


def kernel(x, weight):
    B, L = x.shape
    flat = x.reshape(-1).astype(jnp.int32)
    out = _gather(weight, flat)
    return out.reshape(B, L, DIM)


# speedup vs baseline: 1.5024x; 1.0151x over previous
import functools

import jax
import jax.numpy as jnp
from jax import lax
from jax.experimental import pallas as pl
from jax.experimental.pallas import tpu as pltpu
from jax.experimental.pallas import tpu_sc as plsc

NUM_ROWS = 1_000_000
DIM = 32
B_TOTAL = 4096 * 200

_info = plsc.get_sparse_core_info()
NC, NS = _info.num_cores, _info.num_subcores
NW = NC * NS
PER_W = B_TOTAL // NW
CHUNK = 1600
N_CHUNKS = PER_W // CHUNK
D_BUF = 2


def _make_gather():
    mesh = plsc.VectorSubcoreMesh(core_axis_name="c", subcore_axis_name="s")

    @functools.partial(
        pl.kernel,
        mesh=mesh,
        out_type=jax.ShapeDtypeStruct((B_TOTAL, DIM), jnp.float32),
        scratch_types=[
            pltpu.VMEM((D_BUF, CHUNK), jnp.int32),
            pltpu.VMEM((D_BUF, CHUNK, DIM), jnp.float32),
            pltpu.SemaphoreType.DMA((D_BUF,)),
            pltpu.SemaphoreType.DMA((D_BUF,)),
            pltpu.SemaphoreType.DMA((D_BUF,)),
        ],
        compiler_params=pltpu.CompilerParams(use_tc_tiling_on_sc=False),
    )
    def gather(table_hbm, idx_hbm, out_hbm, idx_v, rows_v, i_sem, g_sem, o_sem):
        wid = lax.axis_index("s") * NC + lax.axis_index("c")
        w_base = wid * PER_W

        def idx_copy(k, b):
            return pltpu.make_async_copy(
                idx_hbm.at[pl.ds(w_base + k * CHUNK, CHUNK)],
                idx_v.at[b], i_sem.at[b])

        def gather_copy(b):
            return pltpu.make_async_copy(
                table_hbm.at[idx_v.at[b]], rows_v.at[b], g_sem.at[b])

        def out_copy(k, b):
            return pltpu.make_async_copy(
                rows_v.at[b],
                out_hbm.at[pl.ds(w_base + k * CHUNK, CHUNK)], o_sem.at[b])

        for b in range(D_BUF):
            idx_copy(b, b).start()

        def body(g, _):
            for b in range(D_BUF):
                k = g * D_BUF + b
                p = (b + 1) % D_BUF
                idx_copy(k, b).wait()

                @pl.when(k >= D_BUF)
                def _():
                    out_copy(k - D_BUF, b).wait()

                gather_copy(b).start()

                @pl.when(k >= 1)
                def _():
                    gather_copy(p).wait()
                    out_copy(k - 1, p).start()

                @pl.when((k >= 1) & (k <= N_CHUNKS - 2))
                def _():
                    idx_copy(k + 1, p).start()

            return 0

        lax.fori_loop(0, N_CHUNKS // D_BUF, body, 0)

        last = N_CHUNKS - 1
        bl = last % D_BUF
        gather_copy(bl).wait()
        out_copy(last, bl).start()
        out_copy(last - 1, (last - 1) % D_BUF).wait()
        out_copy(last, bl).wait()

    return gather


_gather = _make_gather()


def kernel(x, weight):
    B, L = x.shape
    flat = x.reshape(-1).astype(jnp.int32)
    out = _gather(weight, flat)
    return out.reshape(B, L, DIM)
